# Initial kernel scaffold; baseline (speedup 1.0000x reference)
import jax
import jax.numpy as jnp
from jax.experimental import pallas as pl


def kernel(pred, target, batch, x):
    raise NotImplementedError("write your pallas kernel here")


# speedup vs baseline: 6.8148x; 6.8148x over previous
import functools

import jax
import jax.numpy as jnp
from jax import lax
from jax.experimental import pallas as pl
from jax.experimental.pallas import tpu as pltpu
from jax.experimental.pallas import tpu_sc as plsc

_N = 100000
_NSEG = 128
_NW = 16
_ROWS = 50
_CHUNK = _ROWS * 128
_BINS = _NSEG + 16

_LAST_W = (_N - 1) // _CHUNK
_LAST_ROW = ((_N - 1) % _CHUNK) // 128
_LAST_COL = (_N - 1) % 128


def _make_sc_call():
    mesh = plsc.VectorSubcoreMesh(
        core_axis_name="c", subcore_axis_name="s", num_cores=1)

    @functools.partial(
        pl.kernel,
        mesh=mesh,
        out_type=jax.ShapeDtypeStruct((16,), jnp.float32),
        scratch_types=[
            pltpu.VMEM((_CHUNK,), jnp.float32),
            pltpu.VMEM((_CHUNK,), jnp.float32),
            pltpu.VMEM((_ROWS, 128), jnp.int32),
            pltpu.VMEM((128,), jnp.float32),
            pltpu.VMEM((_BINS,), jnp.float32),
            pltpu.VMEM((_BINS,), jnp.float32),
            pltpu.VMEM((16,), jnp.float32),
            pltpu.VMEM((16,), jnp.int32),
            pltpu.VMEM_SHARED((_BINS,), jnp.float32),
            pltpu.VMEM_SHARED((_BINS,), jnp.float32),
        ],
    )
    def sc_loss(pred_hbm, targ_hbm, batch_hbm, out_hbm,
                pred_v, vals_v, batch_v, ones_v, sums_v, cnts_v, out_v,
                ridx_v, sums_sh, cnts_sh):
        w = lax.axis_index("s")

        pltpu.sync_copy(pred_hbm.at[w], pred_v)
        pltpu.sync_copy(targ_hbm.at[w], vals_v)
        pltpu.sync_copy(batch_hbm.at[w], batch_v)

        zeros16 = jnp.zeros((16,), jnp.float32)

        @pl.when(w == 0)
        def _zero_shared():
            for j in range(_BINS // 16):
                sums_v[pl.ds(j * 16, 16)] = zeros16
            pltpu.sync_copy(sums_v, sums_sh)
            pltpu.sync_copy(sums_v, cnts_sh)

        ones16 = jnp.ones((16,), jnp.float32)
        for j in range(8):
            ones_v[pl.ds(j * 16, 16)] = ones16

        def compute_body(i, carry):
            s = pl.ds(pl.multiple_of(i * 16, 16), 16)
            p = pred_v[s]
            t = vals_v[s]
            vals_v[s] = jnp.abs(p * p - t * t)
            return carry
        lax.fori_loop(0, _CHUNK // 16, compute_body, 0, unroll=4)

        plsc.subcore_barrier()

        def scatter_body(j, carry):
            idx = batch_v.at[j]
            src = vals_v.at[pl.ds(pl.multiple_of(j * 128, 128), 128)]
            pltpu.sync_copy(src, sums_sh.at[idx], add=True)
            pltpu.sync_copy(ones_v, cnts_sh.at[idx], add=True)
            return carry
        lax.fori_loop(0, _ROWS, scatter_body, 0)

        plsc.subcore_barrier()

        @pl.when(w == _LAST_W)
        def _finalize():
            pltpu.sync_copy(sums_sh, sums_v)
            pltpu.sync_copy(cnts_sh, cnts_v)
            last_vec = batch_v[_LAST_ROW, pl.ds(_LAST_COL - _LAST_COL % 16, 16)]
            max_b = last_vec[_LAST_COL % 16]
            lane = lax.iota(jnp.int32, 16)
            tot = jnp.zeros((16,), jnp.float32)
            for j in range(_NSEG // 16):
                s = pl.ds(j * 16, 16)
                losses = sums_v[s] / cnts_v[s]
                valid = (lane + (j * 16)) <= max_b
                tot = tot + jnp.where(valid, losses, zeros16)
            red_bin = _NSEG + 1
            ridx_v[...] = (lane * 0) + red_bin
            out_v[...] = tot
            pltpu.sync_copy(out_v, sums_sh.at[ridx_v], add=True)
            pltpu.sync_copy(sums_sh.at[pl.ds(_NSEG, 16)], out_v)
            total_vec = zeros16 + out_v[...][red_bin - _NSEG]
            n_graphs = zeros16 + (max_b + 1).astype(jnp.float32)
            out_v[...] = (total_vec / n_graphs) * 10000.0
            pltpu.sync_copy(out_v, out_hbm)

    return sc_loss


_sc_call = _make_sc_call()


@jax.jit
def kernel(pred, target, batch, x):
    del x
    pad = _NW * _CHUNK - _N
    pred_r = jnp.pad(pred, (0, pad)).reshape(_NW, _CHUNK)
    targ_r = jnp.pad(target, (0, pad)).reshape(_NW, _CHUNK)
    batch_r = jnp.pad(batch, (0, pad), constant_values=_NSEG).reshape(
        _NW, _ROWS, 128)
    out = _sc_call(pred_r, targ_r, batch_r)
    return out[0]
